# Initial kernel scaffold; baseline (speedup 1.0000x reference)
import jax
import jax.numpy as jnp
from jax.experimental import pallas as pl


def kernel(x, edge_index, W1l, b1l, W1r, W2l, b2l, W2r):
    raise NotImplementedError("write your pallas kernel here")


# speedup vs baseline: 4.6924x; 4.6924x over previous
import functools

import jax
import jax.numpy as jnp
from jax import lax
from jax.experimental import pallas as pl
from jax.experimental.pallas import tpu as pltpu
from jax.experimental.pallas import tpu_sc as plsc

N = 10000
E = 320000
NP = 10240
NC = 2
NS = 16
NW = NC * NS
CH = 128
NCHUNK = 80
EPW = NCHUNK * CH
EP = NW * EPW
RPT = NP // NS
CW = 8


def _make_segsum(D, with_cnt):
    mesh = plsc.VectorSubcoreMesh(core_axis_name="c", subcore_axis_name="s")
    out_type = [jax.ShapeDtypeStruct((NC, NP, D), jnp.float32)]
    scratch = [
        pltpu.VMEM((NCHUNK, CH), jnp.int32),
        pltpu.VMEM((NCHUNK, CH), jnp.int32),
        pltpu.VMEM((CH, D), jnp.float32),
        pltpu.VMEM((CH, D), jnp.float32),
        pltpu.VMEM_SHARED((NP, D), jnp.float32),
        pltpu.SemaphoreType.DMA,
        pltpu.SemaphoreType.DMA,
    ]
    if with_cnt:
        out_type.append(jax.ShapeDtypeStruct((NC, NP, CW), jnp.float32))
        scratch += [
            pltpu.VMEM((CH, CW), jnp.float32),
            pltpu.VMEM_SHARED((NP, CW), jnp.float32),
        ]

    def body(*refs):
        if with_cnt:
            (y_hbm, srcs_hbm, dsts_hbm, zrow_hbm, zcnt_hbm, ones_hbm,
             out_hbm, cnt_hbm,
             src_v, dst_v, buf0, buf1, acc, sem0, sem1, ones_v, cacc) = refs
        else:
            (y_hbm, srcs_hbm, dsts_hbm, zrow_hbm,
             out_hbm,
             src_v, dst_v, buf0, buf1, acc, sem0, sem1) = refs

        c = lax.axis_index("c")
        s = lax.axis_index("s")
        wid = s * NC + c

        pltpu.sync_copy(srcs_hbm.at[wid], src_v)
        pltpu.sync_copy(dsts_hbm.at[wid], dst_v)
        pltpu.sync_copy(zrow_hbm, acc.at[pl.ds(s * RPT, RPT)])
        if with_cnt:
            pltpu.sync_copy(zcnt_hbm, cacc.at[pl.ds(s * RPT, RPT)])
            pltpu.sync_copy(ones_hbm, ones_v)
        plsc.subcore_barrier()

        def start(j, buf, sem):
            pltpu.async_copy(y_hbm.at[src_v.at[j]], buf, sem)

        def wait(buf, sem):
            pltpu.make_async_copy(y_hbm.at[src_v.at[0]], buf, sem).wait()

        def scatter(j, buf):
            pltpu.sync_copy(buf, acc.at[dst_v.at[j]], add=True)
            if with_cnt:
                pltpu.sync_copy(ones_v, cacc.at[dst_v.at[j]], add=True)

        start(0, buf0, sem0)
        start(1, buf1, sem1)

        def step(g, carry):
            j0 = 2 * g
            wait(buf0, sem0)
            scatter(j0, buf0)

            @pl.when(j0 + 2 < NCHUNK)
            def _():
                start(j0 + 2, buf0, sem0)

            j1 = j0 + 1
            wait(buf1, sem1)
            scatter(j1, buf1)

            @pl.when(j1 + 2 < NCHUNK)
            def _():
                start(j1 + 2, buf1, sem1)

            return carry

        lax.fori_loop(0, NCHUNK // 2, step, 0)
        plsc.subcore_barrier()

        pltpu.sync_copy(acc.at[pl.ds(s * RPT, RPT)],
                        out_hbm.at[c, pl.ds(s * RPT, RPT)])
        if with_cnt:
            pltpu.sync_copy(cacc.at[pl.ds(s * RPT, RPT)],
                            cnt_hbm.at[c, pl.ds(s * RPT, RPT)])

    return pl.kernel(body, out_type=out_type, mesh=mesh, scratch_types=scratch,
                     compiler_params=pltpu.CompilerParams(use_tc_tiling_on_sc=False),
                     name=f"sc_segsum_d{D}")


_segsum_cnt = _make_segsum(64, with_cnt=True)
_segsum_nc = _make_segsum(64, with_cnt=False)


def _dotT(a, w):
    return lax.dot_general(a, w, (((1,), (1,)), ((), ())),
                           preferred_element_type=jnp.float32)


RB = 1280


def _combine_body(x_ref, s1a_ref, s1b_ref, cnt_ref, w1l_ref, b1l_ref, w1r_ref,
                  w2l_ref, w2r_ref, zl_ref, zr_ref):
    inv = 1.0 / jnp.maximum(cnt_ref[0, :, :1] + cnt_ref[1, :, :1], 1.0)
    meana = (s1a_ref[0] + s1a_ref[1]) * inv
    meanb = (s1b_ref[0] + s1b_ref[1]) * inv
    w1l = w1l_ref[...]
    h = (_dotT(meana, w1l[:, :64]) + _dotT(meanb, w1l[:, 64:])
         + b1l_ref[...] + _dotT(x_ref[...], w1r_ref[...]))
    h = jnp.maximum(h, 0.0)
    zl_ref[...] = _dotT(h, w2l_ref[...])
    zr_ref[...] = _dotT(h, w2r_ref[...])


_combine = pl.pallas_call(
    _combine_body,
    grid=(NP // RB,),
    in_specs=[
        pl.BlockSpec((RB, 128), lambda i: (i, 0)),
        pl.BlockSpec((2, RB, 64), lambda i: (0, i, 0)),
        pl.BlockSpec((2, RB, 64), lambda i: (0, i, 0)),
        pl.BlockSpec((2, RB, CW), lambda i: (0, i, 0)),
        pl.BlockSpec((128, 128), lambda i: (0, 0)),
        pl.BlockSpec((1, 128), lambda i: (0, 0)),
        pl.BlockSpec((128, 128), lambda i: (0, 0)),
        pl.BlockSpec((64, 128), lambda i: (0, 0)),
        pl.BlockSpec((64, 128), lambda i: (0, 0)),
    ],
    out_specs=[
        pl.BlockSpec((RB, 64), lambda i: (i, 0)),
        pl.BlockSpec((RB, 64), lambda i: (i, 0)),
    ],
    out_shape=[
        jax.ShapeDtypeStruct((NP, 64), jnp.float32),
        jax.ShapeDtypeStruct((NP, 64), jnp.float32),
    ],
)


def _final_body(s2_ref, zr_ref, cnt_ref, b2l_ref, out_ref):
    cnt = cnt_ref[0, :, :1] + cnt_ref[1, :, :1]
    o = (s2_ref[0] + s2_ref[1]) / jnp.maximum(cnt, 1.0) + b2l_ref[...] + zr_ref[...]
    m = jnp.max(o, axis=1, keepdims=True)
    lse = jnp.log(jnp.sum(jnp.exp(o - m), axis=1, keepdims=True)) + m
    out_ref[...] = o - lse


_final = pl.pallas_call(
    _final_body,
    grid=(NP // RB,),
    in_specs=[
        pl.BlockSpec((2, RB, 64), lambda i: (0, i, 0)),
        pl.BlockSpec((RB, 64), lambda i: (i, 0)),
        pl.BlockSpec((2, RB, CW), lambda i: (0, i, 0)),
        pl.BlockSpec((1, 64), lambda i: (0, 0)),
    ],
    out_specs=pl.BlockSpec((RB, 64), lambda i: (i, 0)),
    out_shape=jax.ShapeDtypeStruct((NP, 64), jnp.float32),
)


def kernel(x, edge_index, W1l, b1l, W1r, W2l, b2l, W2r):
    src = edge_index[0].astype(jnp.int32)
    dst = edge_index[1].astype(jnp.int32)
    pad = jnp.full((EP - E,), N, jnp.int32)
    srcs = jnp.concatenate([src, pad]).reshape(NW, NCHUNK, CH)
    dsts = jnp.concatenate([dst, pad]).reshape(NW, NCHUNK, CH)
    x_p = jnp.pad(x, ((0, NP - N), (0, 0)))

    zrow64 = jnp.zeros((RPT, 64), jnp.float32)
    zcnt = jnp.zeros((RPT, CW), jnp.float32)
    ones = jnp.ones((CH, CW), jnp.float32)

    s1a, cnt1 = _segsum_cnt(x_p[:, :64], srcs, dsts, zrow64, zcnt, ones)
    (s1b,) = _segsum_nc(x_p[:, 64:], srcs, dsts, zrow64)
    zl, zr = _combine(x_p, s1a, s1b, cnt1, W1l, b1l.reshape(1, -1), W1r, W2l, W2r)
    (s2,) = _segsum_nc(zl, srcs, dsts, zrow64)
    out = _final(s2, zr, cnt1, b2l.reshape(1, -1))
    return out[:N]
